# baseline (speedup 1.0000x reference)
import functools

import jax
import jax.numpy as jnp
from jax import lax
from jax.experimental import pallas as pl
from jax.experimental.pallas import tpu as pltpu
from jax.experimental.pallas import tpu_sc as plsc

NC, NS, L = 2, 16, 16
NW = NC * NS
B, LPATH, D = 4096, 200, 100
DP = 128
BPW = B // NW
IDXROWS = 2 * BPW


def _sc_gather_max(x3, E128):
  mesh = plsc.VectorSubcoreMesh(
      core_axis_name="c", subcore_axis_name="s", num_cores=NC, num_subcores=NS)

  @functools.partial(
      pl.kernel,
      out_type=jax.ShapeDtypeStruct((NW, 8, DP), jnp.float32),
      mesh=mesh,
      scratch_types=[
          pltpu.VMEM((IDXROWS, DP), jnp.int32),
          pltpu.VMEM((512, DP), jnp.float32),
          pltpu.VMEM((8, DP), jnp.float32),
          pltpu.SemaphoreType.DMA,
          pltpu.SemaphoreType.DMA,
      ],
  )
  def k(x_hbm, tbl_hbm, out_hbm, idx_v, gbuf, mstage, sem0, sem1):
    wid = lax.axis_index("s") * NC + lax.axis_index("c")
    base = wid * IDXROWS
    pltpu.sync_copy(x_hbm.at[pl.ds(base, IDXROWS)], idx_v)

    def start(b, rowbase, sem):
      for h in range(2):
        pltpu.async_copy(tbl_hbm.at[idx_v.at[2 * b + h, pl.ds(0, D)]],
                         gbuf.at[pl.ds(rowbase + h * 128, D)], sem)

    def wait(rowbase, sem):
      for h in range(2):
        pltpu.make_async_copy(tbl_hbm.at[idx_v.at[0, pl.ds(0, D)]],
                              gbuf.at[pl.ds(rowbase + h * 128, D)], sem).wait()

    def reduce_half(rowbase, acc):
      def rbody(r, a):
        row = rowbase + r
        return tuple(
            a[j] + gbuf[row, pl.ds(84 if j == 6 else j * 16, L)]
            for j in range(7))
      return lax.fori_loop(0, D, rbody, acc)

    def reduce_block(rowbase, m):
      zero = jnp.zeros((L,), jnp.float32)
      acc = reduce_half(rowbase, (zero,) * 7)
      acc = reduce_half(rowbase + 128, acc)
      out = []
      for j in range(7):
        s = acc[j]
        h = jnp.maximum(s, 0.0) + jnp.float32(0.01) * jnp.minimum(s, 0.0)
        out.append(jnp.maximum(m[j], h))
      return tuple(out)

    start(0, 0, sem0)

    def body(i, m):
      start(2 * i + 1, 256, sem1)
      wait(0, sem0)
      m = reduce_block(0, m)

      @pl.when(i < BPW // 2 - 1)
      def _():
        start(2 * i + 2, 0, sem0)
      wait(256, sem1)
      m = reduce_block(256, m)
      return m

    neg = jnp.full((L,), -3.0e38, jnp.float32)
    m = lax.fori_loop(0, BPW // 2, body, (neg,) * 7)
    for j in range(7):
      mstage[0, pl.ds(j * 16, L)] = m[j]
    pltpu.sync_copy(mstage, out_hbm.at[wid])

  return k(x3, E128)


def _tc_classifier(wpart, w_out, b_out, y):
  def body(wp_ref, wo_ref, bo_ref, y_ref, pred_ref, loss_ref):
    wp = wp_ref[...][:, 0, :]
    colmax = jnp.max(wp, axis=0)
    pmax = jnp.concatenate([colmax[:96], colmax[108:112]])
    wo = wo_ref[...]
    logits = jnp.sum(wo * pmax[None, :], axis=1) + bo_ref[...]
    mx = jnp.max(logits)
    e = jnp.exp(logits - mx)
    pred = e / jnp.sum(e)
    m2 = jnp.max(pred)
    lse = m2 + jnp.log(jnp.sum(jnp.exp(pred - m2)))
    logp = pred - lse
    loss = -jnp.sum(y_ref[...] * logp)
    pred_ref[...] = pred[None, :]
    loss_ref[...] = loss[None, None]

  return pl.pallas_call(
      body,
      out_shape=(
          jax.ShapeDtypeStruct((1, 4), jnp.float32),
          jax.ShapeDtypeStruct((1, 1), jnp.float32),
      ),
  )(wpart, w_out, b_out, y)


def kernel(x, y, E_td, w_out, b_out):
  x3 = jnp.pad(x.reshape(2 * B, D), ((0, 0), (0, DP - D)))
  E128 = jnp.pad(E_td, ((0, 0), (0, DP - D)))
  wpart = _sc_gather_max(x3, E128)
  pred, loss = _tc_classifier(wpart, w_out, b_out, y)
  return (pred.reshape(4), loss.reshape(()))


# speedup vs baseline: 1.8995x; 1.8995x over previous
import functools

import jax
import jax.numpy as jnp
from jax import lax
from jax.experimental import pallas as pl
from jax.experimental.pallas import tpu as pltpu
from jax.experimental.pallas import tpu_sc as plsc

NC, NS, L = 2, 16, 16
NW = NC * NS
B, LPATH, D = 4096, 200, 100
DP = 128
BPW = B // NW
IDXROWS = 2 * BPW


def _sc_gather_max(x3, E128):
  mesh = plsc.VectorSubcoreMesh(
      core_axis_name="c", subcore_axis_name="s", num_cores=NC, num_subcores=NS)

  @functools.partial(
      pl.kernel,
      out_type=jax.ShapeDtypeStruct((NW, 8, DP), jnp.float32),
      mesh=mesh,
      scratch_types=[
          pltpu.VMEM((IDXROWS, DP), jnp.int32),
          pltpu.VMEM((512, DP), jnp.float32),
          pltpu.VMEM((8, DP), jnp.float32),
          pltpu.SemaphoreType.DMA,
          pltpu.SemaphoreType.DMA,
      ],
  )
  def k(x_hbm, tbl_hbm, out_hbm, idx_v, gbuf, mstage, sem0, sem1):
    wid = lax.axis_index("s") * NC + lax.axis_index("c")
    base = wid * IDXROWS
    pltpu.sync_copy(x_hbm.at[pl.ds(base, IDXROWS)], idx_v)

    def start(b, rowbase, sem):
      for h in range(2):
        pltpu.async_copy(tbl_hbm.at[idx_v.at[2 * b + h, pl.ds(0, D)]],
                         gbuf.at[pl.ds(rowbase + h * 128, D)], sem)

    def wait(rowbase, sem):
      for h in range(2):
        pltpu.make_async_copy(tbl_hbm.at[idx_v.at[0, pl.ds(0, D)]],
                              gbuf.at[pl.ds(rowbase + h * 128, D)], sem).wait()

    def reduce_half(rowbase, acc):
      def rbody(r, a):
        row = rowbase + r
        return tuple(
            a[j] + gbuf[row, pl.ds(84 if j == 6 else j * 16, L)]
            for j in range(7))
      return lax.fori_loop(0, D, rbody, acc)

    def reduce_block(rowbase, m):
      zero = jnp.zeros((L,), jnp.float32)
      acc = reduce_half(rowbase, (zero,) * 7)
      acc = reduce_half(rowbase + 128, acc)
      out = []
      for j in range(7):
        s = acc[j]
        h = jnp.maximum(s, 0.0) + jnp.float32(0.01) * jnp.minimum(s, 0.0)
        out.append(jnp.maximum(m[j], h))
      return tuple(out)

    start(0, 0, sem0)

    def body(i, m):
      start(2 * i + 1, 256, sem1)
      wait(0, sem0)
      m = reduce_block(0, m)

      @pl.when(i < BPW // 2 - 1)
      def _():
        start(2 * i + 2, 0, sem0)
      wait(256, sem1)
      m = reduce_block(256, m)
      return m

    neg = jnp.full((L,), -3.0e38, jnp.float32)
    m = lax.fori_loop(0, BPW // 2, body, (neg,) * 7)
    for j in range(7):
      mstage[0, pl.ds(j * 16, L)] = m[j]
    pltpu.sync_copy(mstage, out_hbm.at[wid])

  return k(x3, E128)


_PADR = 2000


def _tc_pad_table(E_td):
  V = E_td.shape[0]

  def body(in_ref, out_ref):
    out_ref[:, :D] = in_ref[...]
    out_ref[:, D:] = jnp.zeros((_PADR, DP - D), jnp.float32)

  return pl.pallas_call(
      body,
      grid=(V // _PADR,),
      in_specs=[pl.BlockSpec((_PADR, D), lambda i: (i, 0))],
      out_specs=pl.BlockSpec((_PADR, DP), lambda i: (i, 0)),
      out_shape=jax.ShapeDtypeStruct((V, DP), jnp.float32),
  )(E_td)


def _tc_classifier(wpart, w_out, b_out, y):
  def body(wp_ref, wo_ref, bo_ref, y_ref, pred_ref, loss_ref):
    wp = wp_ref[...][:, 0, :]
    colmax = jnp.max(wp, axis=0)
    pmax = jnp.concatenate([colmax[:96], colmax[108:112]])
    wo = wo_ref[...]
    logits = jnp.sum(wo * pmax[None, :], axis=1) + bo_ref[...]
    mx = jnp.max(logits)
    e = jnp.exp(logits - mx)
    pred = e / jnp.sum(e)
    m2 = jnp.max(pred)
    lse = m2 + jnp.log(jnp.sum(jnp.exp(pred - m2)))
    logp = pred - lse
    loss = -jnp.sum(y_ref[...] * logp)
    pred_ref[...] = pred[None, :]
    loss_ref[...] = loss[None, None]

  return pl.pallas_call(
      body,
      out_shape=(
          jax.ShapeDtypeStruct((1, 4), jnp.float32),
          jax.ShapeDtypeStruct((1, 1), jnp.float32),
      ),
  )(wpart, w_out, b_out, y)


def kernel(x, y, E_td, w_out, b_out):
  x3 = jnp.pad(x.reshape(2 * B, D), ((0, 0), (0, DP - D)))
  E128 = _tc_pad_table(E_td)
  wpart = _sc_gather_max(x3, E128)
  pred, loss = _tc_classifier(wpart, w_out, b_out, y)
  return (pred.reshape(4), loss.reshape(()))


# speedup vs baseline: 2.2682x; 1.1941x over previous
import functools

import jax
import jax.numpy as jnp
from jax import lax
from jax.experimental import pallas as pl
from jax.experimental.pallas import tpu as pltpu
from jax.experimental.pallas import tpu_sc as plsc

NC, NS, L = 2, 16, 16
NW = NC * NS
B, LPATH, D = 4096, 200, 100
DP = 128
BPW = B // NW
IDXROWS = 2 * BPW


def _sc_gather_max(x3, E128):
  mesh = plsc.VectorSubcoreMesh(
      core_axis_name="c", subcore_axis_name="s", num_cores=NC, num_subcores=NS)

  @functools.partial(
      pl.kernel,
      out_type=jax.ShapeDtypeStruct((NW, 8, DP), jnp.float32),
      mesh=mesh,
      scratch_types=[
          pltpu.VMEM((IDXROWS, DP), jnp.int32),
          pltpu.VMEM((512, DP), jnp.float32),
          pltpu.VMEM((8, DP), jnp.float32),
          pltpu.SemaphoreType.DMA,
          pltpu.SemaphoreType.DMA,
      ],
  )
  def k(x_hbm, tbl_hbm, out_hbm, idx_v, gbuf, mstage, sem0, sem1):
    wid = lax.axis_index("s") * NC + lax.axis_index("c")
    base = wid * IDXROWS
    pltpu.sync_copy(x_hbm.at[pl.ds(base, IDXROWS)], idx_v)

    def start(b, rowbase, sem):
      for h in range(2):
        pltpu.async_copy(tbl_hbm.at[idx_v.at[2 * b + h, pl.ds(0, D)]],
                         gbuf.at[pl.ds(rowbase + h * 128, D)], sem)

    def wait(rowbase, sem):
      for h in range(2):
        pltpu.make_async_copy(tbl_hbm.at[idx_v.at[0, pl.ds(0, D)]],
                              gbuf.at[pl.ds(rowbase + h * 128, D)], sem).wait()

    def reduce_half(rowbase, acc):
      def rbody(r, a):
        row = rowbase + r
        return tuple(
            a[j] + gbuf[row, pl.ds(84 if j == 6 else j * 16, L)]
            for j in range(7))
      return lax.fori_loop(0, D, rbody, acc)

    def reduce_block(rowbase, m):
      zero = jnp.zeros((L,), jnp.float32)
      acc = reduce_half(rowbase, (zero,) * 7)
      acc = reduce_half(rowbase + 128, acc)
      out = []
      for j in range(7):
        s = acc[j]
        h = jnp.maximum(s, 0.0) + jnp.float32(0.01) * jnp.minimum(s, 0.0)
        out.append(jnp.maximum(m[j], h))
      return tuple(out)

    start(0, 0, sem0)

    def body(i, m):
      start(2 * i + 1, 256, sem1)
      wait(0, sem0)
      m = reduce_block(0, m)

      @pl.when(i < BPW // 2 - 1)
      def _():
        start(2 * i + 2, 0, sem0)
      wait(256, sem1)
      m = reduce_block(256, m)
      return m

    neg = jnp.full((L,), -3.0e38, jnp.float32)
    m = lax.fori_loop(0, BPW // 2, body, (neg,) * 7)
    for j in range(7):
      mstage[0, pl.ds(j * 16, L)] = m[j]
    pltpu.sync_copy(mstage, out_hbm.at[wid])

  return k(x3, E128)


_PADR = 8000


def _tc_pad_table(E_td):
  V = E_td.shape[0]

  def body(in_ref, out_ref):
    out_ref[:, :D] = in_ref[...]
    out_ref[:, D:] = jnp.zeros((_PADR, DP - D), jnp.float32)

  return pl.pallas_call(
      body,
      grid=(V // _PADR,),
      in_specs=[pl.BlockSpec((_PADR, D), lambda i: (i, 0))],
      out_specs=pl.BlockSpec((_PADR, DP), lambda i: (i, 0)),
      out_shape=jax.ShapeDtypeStruct((V, DP), jnp.float32),
  )(E_td)


def _tc_classifier(wpart, w_out, b_out, y):
  def body(wp_ref, wo_ref, bo_ref, y_ref, pred_ref, loss_ref):
    wp = wp_ref[...][:, 0, :]
    colmax = jnp.max(wp, axis=0)
    pmax = jnp.concatenate([colmax[:96], colmax[108:112]])
    wo = wo_ref[...]
    logits = jnp.sum(wo * pmax[None, :], axis=1) + bo_ref[...]
    mx = jnp.max(logits)
    e = jnp.exp(logits - mx)
    pred = e / jnp.sum(e)
    m2 = jnp.max(pred)
    lse = m2 + jnp.log(jnp.sum(jnp.exp(pred - m2)))
    logp = pred - lse
    loss = -jnp.sum(y_ref[...] * logp)
    pred_ref[...] = pred[None, :]
    loss_ref[...] = loss[None, None]

  return pl.pallas_call(
      body,
      out_shape=(
          jax.ShapeDtypeStruct((1, 4), jnp.float32),
          jax.ShapeDtypeStruct((1, 1), jnp.float32),
      ),
  )(wpart, w_out, b_out, y)


def kernel(x, y, E_td, w_out, b_out):
  x3 = jnp.pad(x.reshape(2 * B, D), ((0, 0), (0, DP - D)))
  E128 = _tc_pad_table(E_td)
  wpart = _sc_gather_max(x3, E128)
  pred, loss = _tc_classifier(wpart, w_out, b_out, y)
  return (pred.reshape(4), loss.reshape(()))


# speedup vs baseline: 2.2744x; 1.0027x over previous
import functools

import jax
import jax.numpy as jnp
from jax import lax
from jax.experimental import pallas as pl
from jax.experimental.pallas import tpu as pltpu
from jax.experimental.pallas import tpu_sc as plsc

NC, NS, L = 2, 16, 16
NW = NC * NS
B, LPATH, D = 4096, 200, 100
DP = 128
BPW = B // NW
IDXROWS = 2 * BPW


def _sc_gather_max(x3, E128):
  mesh = plsc.VectorSubcoreMesh(
      core_axis_name="c", subcore_axis_name="s", num_cores=NC, num_subcores=NS)

  @functools.partial(
      pl.kernel,
      out_type=jax.ShapeDtypeStruct((NW, 8, DP), jnp.float32),
      mesh=mesh,
      scratch_types=[
          pltpu.VMEM((IDXROWS, DP), jnp.int32),
          pltpu.VMEM((512, DP), jnp.float32),
          pltpu.VMEM((8, DP), jnp.float32),
          pltpu.SemaphoreType.DMA,
          pltpu.SemaphoreType.DMA,
      ],
  )
  def k(x_hbm, tbl_hbm, out_hbm, idx_v, gbuf, mstage, sem0, sem1):
    wid = lax.axis_index("s") * NC + lax.axis_index("c")
    base = wid * IDXROWS
    pltpu.sync_copy(x_hbm.at[pl.ds(base, IDXROWS)], idx_v)

    def start(b, rowbase, sem):
      for h in range(2):
        pltpu.async_copy(tbl_hbm.at[idx_v.at[2 * b + h, pl.ds(0, D)]],
                         gbuf.at[pl.ds(rowbase + h * 128, D)], sem)

    def wait(rowbase, sem):
      for h in range(2):
        pltpu.make_async_copy(tbl_hbm.at[idx_v.at[0, pl.ds(0, D)]],
                              gbuf.at[pl.ds(rowbase + h * 128, D)], sem).wait()

    def reduce_half(rowbase, acc):
      def rbody(r, a):
        row = rowbase + r
        return tuple(
            a[j] + gbuf[row, pl.ds(84 if j == 6 else j * 16, L)]
            for j in range(7))
      return lax.fori_loop(0, D, rbody, acc)

    def reduce_block(rowbase, m):
      zero = jnp.zeros((L,), jnp.float32)
      acc = reduce_half(rowbase, (zero,) * 7)
      acc = reduce_half(rowbase + 128, acc)
      out = []
      for j in range(7):
        s = acc[j]
        h = jnp.maximum(s, 0.0) + jnp.float32(0.01) * jnp.minimum(s, 0.0)
        out.append(jnp.maximum(m[j], h))
      return tuple(out)

    start(0, 0, sem0)

    def body(i, m):
      start(2 * i + 1, 256, sem1)
      wait(0, sem0)
      m = reduce_block(0, m)

      @pl.when(i < BPW // 2 - 1)
      def _():
        start(2 * i + 2, 0, sem0)
      wait(256, sem1)
      m = reduce_block(256, m)
      return m

    neg = jnp.full((L,), -3.0e38, jnp.float32)
    m = lax.fori_loop(0, BPW // 2, body, (neg,) * 7)
    for j in range(7):
      mstage[0, pl.ds(j * 16, L)] = m[j]
    pltpu.sync_copy(mstage, out_hbm.at[wid])

  return k(x3, E128)


_PADR = 20000


def _tc_pad_table(E_td):
  V = E_td.shape[0]

  def body(in_ref, out_ref):
    out_ref[:, :D] = in_ref[...]
    out_ref[:, D:] = jnp.zeros((_PADR, DP - D), jnp.float32)

  return pl.pallas_call(
      body,
      grid=(V // _PADR,),
      in_specs=[pl.BlockSpec((_PADR, D), lambda i: (i, 0))],
      out_specs=pl.BlockSpec((_PADR, DP), lambda i: (i, 0)),
      out_shape=jax.ShapeDtypeStruct((V, DP), jnp.float32),
  )(E_td)


def _tc_classifier(wpart, w_out, b_out, y):
  def body(wp_ref, wo_ref, bo_ref, y_ref, pred_ref, loss_ref):
    wp = wp_ref[...][:, 0, :]
    colmax = jnp.max(wp, axis=0)
    pmax = jnp.concatenate([colmax[:96], colmax[108:112]])
    wo = wo_ref[...]
    logits = jnp.sum(wo * pmax[None, :], axis=1) + bo_ref[...]
    mx = jnp.max(logits)
    e = jnp.exp(logits - mx)
    pred = e / jnp.sum(e)
    m2 = jnp.max(pred)
    lse = m2 + jnp.log(jnp.sum(jnp.exp(pred - m2)))
    logp = pred - lse
    loss = -jnp.sum(y_ref[...] * logp)
    pred_ref[...] = pred[None, :]
    loss_ref[...] = loss[None, None]

  return pl.pallas_call(
      body,
      out_shape=(
          jax.ShapeDtypeStruct((1, 4), jnp.float32),
          jax.ShapeDtypeStruct((1, 1), jnp.float32),
      ),
  )(wpart, w_out, b_out, y)


def kernel(x, y, E_td, w_out, b_out):
  x3 = jnp.pad(x.reshape(2 * B, D), ((0, 0), (0, DP - D)))
  E128 = _tc_pad_table(E_td)
  wpart = _sc_gather_max(x3, E128)
  pred, loss = _tc_classifier(wpart, w_out, b_out, y)
  return (pred.reshape(4), loss.reshape(()))
